# baseline (speedup 1.0000x reference)
import functools

import jax
import jax.numpy as jnp
from jax import lax
from jax.experimental import pallas as pl
from jax.experimental.pallas import tpu as pltpu
from jax.experimental.pallas import tpu_sc as plsc

DIM = 64
L = 16
NC, NS = 2, 16
NW = NC * NS
CHUNK = 128


def _cosine_body(batch, oid_hbm, eid_hbm, mentors_hbm, mentees_hbm, out_hbm,
                 oid_v, eid_v, orows_v, erows_v, out_v, sem):
    bpw = batch // NW
    nchunk = bpw // CHUNK
    wid = lax.axis_index("s") * NC + lax.axis_index("c")
    cbase = wid * nchunk

    pltpu.sync_copy(oid_hbm.at[pl.ds(cbase, nchunk)], oid_v)
    pltpu.sync_copy(eid_hbm.at[pl.ds(cbase, nchunk)], eid_v)

    copies = []
    for c in range(nchunk):
        copies.append(pltpu.async_copy(mentors_hbm.at[oid_v.at[c]], orows_v.at[c], sem))
        copies.append(pltpu.async_copy(mentees_hbm.at[eid_v.at[c]], erows_v.at[c], sem))
    for cp in copies:
        cp.wait()

    lane = lax.iota(jnp.int32, L)
    nseg = DIM // L

    def group(c, j):
        dotv = jnp.zeros((L,), jnp.float32)
        pv = jnp.zeros((L,), jnp.float32)
        for r in range(L):
            row = j * L + r
            dot = jnp.zeros((L,), jnp.float32)
            on = jnp.zeros((L,), jnp.float32)
            en = jnp.zeros((L,), jnp.float32)
            for k in range(nseg):
                o = orows_v[c, row, pl.ds(k * L, L)]
                e = erows_v[c, row, pl.ds(k * L, L)]
                dot = dot + o * e
                on = on + o * o
                en = en + e * e
            sdot = jnp.sum(dot)
            sp = jnp.sum(on) * jnp.sum(en)
            dotv = jnp.where(lane == r, sdot, dotv)
            pv = jnp.where(lane == r, sp, pv)
        yi = jnp.int32(0x5F3759DF) - lax.shift_right_logical(
            plsc.bitcast(pv, jnp.int32), 1)
        y = plsc.bitcast(yi, jnp.float32)
        xh = pv * jnp.float32(0.5)
        for _ in range(3):
            y = y * (jnp.float32(1.5) - xh * y * y)
        out_v[c, pl.ds(j * L, L)] = dotv * y

    for c in range(nchunk):
        lax.fori_loop(0, CHUNK // L, lambda j, _, c=c: (group(c, j), 0)[1], 0)

    pltpu.sync_copy(out_v, out_hbm.at[pl.ds(cbase, nchunk)])


def kernel(o_id, e_id, mentors, mentees):
    batch = o_id.shape[0]
    bpw = batch // NW
    nchunk = bpw // CHUNK
    oid2 = o_id.reshape(batch // CHUNK, CHUNK)
    eid2 = e_id.reshape(batch // CHUNK, CHUNK)

    mesh = plsc.VectorSubcoreMesh(core_axis_name="c", subcore_axis_name="s",
                                  num_cores=NC, num_subcores=NS)
    call = pl.kernel(
        functools.partial(_cosine_body, batch),
        out_type=jax.ShapeDtypeStruct((batch // CHUNK, CHUNK), jnp.float32),
        mesh=mesh,
        compiler_params=pltpu.CompilerParams(needs_layout_passes=False,
                                             use_tc_tiling_on_sc=False),
        scratch_types=[
            pltpu.VMEM((nchunk, CHUNK), jnp.int32),
            pltpu.VMEM((nchunk, CHUNK), jnp.int32),
            pltpu.VMEM((nchunk, CHUNK, DIM), jnp.float32),
            pltpu.VMEM((nchunk, CHUNK, DIM), jnp.float32),
            pltpu.VMEM((nchunk, CHUNK), jnp.float32),
            pltpu.SemaphoreType.DMA,
        ],
    )
    out2 = call(oid2, eid2, mentors, mentees)
    return out2.reshape(batch)
